# Initial kernel scaffold; baseline (speedup 1.0000x reference)
import jax
import jax.numpy as jnp
from jax.experimental import pallas as pl


def kernel(inputs, embedding, embedding_unnormalized):
    raise NotImplementedError("write your pallas kernel here")


# speedup vs baseline: 1.3664x; 1.3664x over previous
import jax
import jax.numpy as jnp
from jax import lax
from jax.experimental import pallas as pl

B, D, T = 32, 64, 576
K = 1024
N = B * T
EPS = 1e-12


def _vq_kernel(x_ref, emb_ref, embu_ref, quant_ref, idx_ref, loss_ref):
    b = pl.program_id(0)

    x = x_ref[0]
    emb = emb_ref[...]
    embu = embu_ref[...]

    norm = jnp.sqrt(jnp.sum(x * x, axis=0, keepdims=True))
    xn = x / jnp.maximum(norm, EPS)

    dist = lax.dot_general(
        emb, xn, (((1,), (0,)), ((), ())),
        preferred_element_type=jnp.float32,
    )

    maxval = jnp.max(dist, axis=0, keepdims=True)
    code_iota = lax.broadcasted_iota(jnp.int32, (K, T), 0)
    idx = jnp.min(jnp.where(dist >= maxval, code_iota, K), axis=0)
    idx_ref[0, 0] = idx

    onehot = (code_iota == idx[None, :]).astype(jnp.float32)
    quant = lax.dot_general(
        emb, onehot, (((0,), (0,)), ((), ())),
        preferred_element_type=jnp.float32,
        precision=lax.Precision.HIGHEST,
    )
    quant_ref[0] = quant

    qu = lax.dot_general(
        embu, onehot, (((0,), (0,)), ((), ())),
        preferred_element_type=jnp.float32,
        precision=lax.Precision.HIGHEST,
    )
    diff = x - qu
    part = jnp.sum(diff * diff).reshape(1, 1)

    @pl.when(b == 0)
    def _init():
        loss_ref[...] = part

    @pl.when(b != 0)
    def _acc():
        loss_ref[...] += part


def kernel(inputs, embedding, embedding_unnormalized):
    quant, idx3, loss_sum = pl.pallas_call(
        _vq_kernel,
        grid=(B,),
        in_specs=[
            pl.BlockSpec((1, D, T), lambda b: (b, 0, 0)),
            pl.BlockSpec((K, D), lambda b: (0, 0)),
            pl.BlockSpec((K, D), lambda b: (0, 0)),
        ],
        out_specs=[
            pl.BlockSpec((1, D, T), lambda b: (b, 0, 0)),
            pl.BlockSpec((1, 1, T), lambda b: (b, 0, 0)),
            pl.BlockSpec((1, 1), lambda b: (0, 0)),
        ],
        out_shape=[
            jax.ShapeDtypeStruct((B, D, T), jnp.float32),
            jax.ShapeDtypeStruct((B, 1, T), jnp.int32),
            jax.ShapeDtypeStruct((1, 1), jnp.float32),
        ],
    )(inputs, embedding, embedding_unnormalized)

    loss = loss_sum[0, 0] / jnp.float32(N * D)
    encoding_indices = idx3.reshape(N)
    return (quant, loss, loss, encoding_indices)


# speedup vs baseline: 3.3269x; 2.4348x over previous
import jax
import jax.numpy as jnp
from jax import lax
from jax.experimental import pallas as pl

B, D, T = 32, 64, 576
K = 1024
N = B * T
EPS = 1e-12


def _vq_kernel(x_ref, emb_ref, cat_ref, quant_ref, idx_ref, loss_ref):
    b = pl.program_id(0)

    @pl.when(b == 0)
    def _init():
        loss_ref[...] = jnp.zeros((1, 1), jnp.float32)

    x = x_ref[0]
    emb = emb_ref[...]
    cat = cat_ref[...]

    norm = jnp.sqrt(jnp.sum(x * x, axis=0, keepdims=True))
    xn = x / jnp.maximum(norm, EPS)

    dist = lax.dot_general(
        emb, xn, (((1,), (0,)), ((), ())),
        preferred_element_type=jnp.float32,
    )

    maxval = jnp.max(dist, axis=0, keepdims=True)
    iota_f = lax.broadcasted_iota(jnp.int32, (K, T), 0).astype(jnp.float32)
    idxf = jnp.min(jnp.where(dist >= maxval, iota_f, float(K)), axis=0)
    idx_ref[0, 0] = idxf.astype(jnp.int32)

    onehot = (iota_f == idxf[None, :]).astype(jnp.float32)
    combo = lax.dot_general(
        cat, onehot, (((0,), (0,)), ((), ())),
        preferred_element_type=jnp.float32,
    )
    quant_ref[0] = combo[:D]
    diff = x - combo[D:]
    loss_ref[...] += jnp.sum(diff * diff).reshape(1, 1)


def kernel(inputs, embedding, embedding_unnormalized):
    cat = jnp.concatenate([embedding, embedding_unnormalized], axis=1)

    quant, idx3, loss_sum = pl.pallas_call(
        _vq_kernel,
        grid=(B,),
        in_specs=[
            pl.BlockSpec((1, D, T), lambda b: (b, 0, 0)),
            pl.BlockSpec((K, D), lambda b: (0, 0)),
            pl.BlockSpec((K, 2 * D), lambda b: (0, 0)),
        ],
        out_specs=[
            pl.BlockSpec((1, D, T), lambda b: (b, 0, 0)),
            pl.BlockSpec((1, 1, T), lambda b: (b, 0, 0)),
            pl.BlockSpec((1, 1), lambda b: (0, 0)),
        ],
        out_shape=[
            jax.ShapeDtypeStruct((B, D, T), jnp.float32),
            jax.ShapeDtypeStruct((B, 1, T), jnp.int32),
            jax.ShapeDtypeStruct((1, 1), jnp.float32),
        ],
    )(inputs, embedding, cat)

    loss = loss_sum[0, 0] / jnp.float32(N * D)
    encoding_indices = idx3.reshape(N)
    return (quant, loss, loss, encoding_indices)


# speedup vs baseline: 4.3499x; 1.3075x over previous
import jax
import jax.numpy as jnp
from jax import lax
from jax.experimental import pallas as pl

B, D, T = 32, 64, 576
K = 1024
N = B * T
EPS = 1e-12
BB = 2
W = BB * T


def _vq_kernel(x_ref, emb_ref, cat_ref, quant_ref, idx_ref, loss_ref):
    g = pl.program_id(0)

    @pl.when(g == 0)
    def _init():
        loss_ref[...] = jnp.zeros((1, 1), jnp.float32)

    x = jnp.concatenate([x_ref[i] for i in range(BB)], axis=1)
    emb = emb_ref[...]
    cat = cat_ref[...]

    norm = jnp.sqrt(jnp.sum(x * x, axis=0, keepdims=True))
    xn = x / jnp.maximum(norm, EPS)

    dist = lax.dot_general(
        emb, xn, (((1,), (0,)), ((), ())),
        preferred_element_type=jnp.float32,
    )

    maxval = jnp.max(dist, axis=0, keepdims=True)
    iota_f = lax.broadcasted_iota(jnp.int32, (K, W), 0).astype(jnp.float32)
    idxf = jnp.min(jnp.where(dist >= maxval, iota_f, float(K)), axis=0)

    onehot = (iota_f == idxf[None, :]).astype(jnp.float32)
    combo = lax.dot_general(
        cat, onehot, (((0,), (0,)), ((), ())),
        preferred_element_type=jnp.float32,
    )

    idx = idxf.astype(jnp.int32)
    for i in range(BB):
        idx_ref[i, 0] = idx[i * T:(i + 1) * T]
        quant_ref[i] = combo[:D, i * T:(i + 1) * T]
    diff = x - combo[D:]
    loss_ref[...] += jnp.sum(diff * diff).reshape(1, 1)


def kernel(inputs, embedding, embedding_unnormalized):
    cat = jnp.concatenate([embedding, embedding_unnormalized], axis=1)

    quant, idx3, loss_sum = pl.pallas_call(
        _vq_kernel,
        grid=(B // BB,),
        in_specs=[
            pl.BlockSpec((BB, D, T), lambda g: (g, 0, 0)),
            pl.BlockSpec((K, D), lambda g: (0, 0)),
            pl.BlockSpec((K, 2 * D), lambda g: (0, 0)),
        ],
        out_specs=[
            pl.BlockSpec((BB, D, T), lambda g: (g, 0, 0)),
            pl.BlockSpec((BB, 1, T), lambda g: (g, 0, 0)),
            pl.BlockSpec((1, 1), lambda g: (0, 0)),
        ],
        out_shape=[
            jax.ShapeDtypeStruct((B, D, T), jnp.float32),
            jax.ShapeDtypeStruct((B, 1, T), jnp.int32),
            jax.ShapeDtypeStruct((1, 1), jnp.float32),
        ],
    )(inputs, embedding, cat)

    loss = loss_sum[0, 0] / jnp.float32(N * D)
    encoding_indices = idx3.reshape(N)
    return (quant, loss, loss, encoding_indices)


# speedup vs baseline: 4.3567x; 1.0016x over previous
import jax
import jax.numpy as jnp
from jax import lax
from jax.experimental import pallas as pl
from jax.experimental.pallas import tpu as pltpu

B, D, T = 32, 64, 576
K = 1024
N = B * T
EPS = 1e-12
BB = 2
W = BB * T


def _vq_kernel(x_ref, emb_ref, embu_ref, quant_ref, idx_ref, loss_ref, cat_ref):
    g = pl.program_id(0)

    @pl.when(g == 0)
    def _init():
        loss_ref[...] = jnp.zeros((1, 1), jnp.float32)
        cat_ref[:, :D] = emb_ref[...].astype(jnp.bfloat16)
        cat_ref[:, D:] = embu_ref[...].astype(jnp.bfloat16)

    x = jnp.concatenate([x_ref[i] for i in range(BB)], axis=1)
    emb = emb_ref[...]

    norm = jnp.sqrt(jnp.sum(x * x, axis=0, keepdims=True))
    xn = x / jnp.maximum(norm, EPS)

    dist = lax.dot_general(
        emb, xn, (((1,), (0,)), ((), ())),
        preferred_element_type=jnp.float32,
    )

    maxval = jnp.max(dist, axis=0, keepdims=True)
    iota_f = lax.broadcasted_iota(jnp.int32, (K, W), 0).astype(jnp.float32)
    idxf = jnp.min(jnp.where(dist >= maxval, iota_f, float(K)), axis=0)

    onehot = (iota_f == idxf[None, :]).astype(jnp.bfloat16)
    combo = lax.dot_general(
        cat_ref[...], onehot, (((0,), (0,)), ((), ())),
        preferred_element_type=jnp.float32,
    )

    idx = idxf.astype(jnp.int32)
    for i in range(BB):
        idx_ref[i, 0] = idx[i * T:(i + 1) * T]
        quant_ref[i] = combo[:D, i * T:(i + 1) * T]
    diff = x - combo[D:]
    loss_ref[...] += jnp.sum(diff * diff).reshape(1, 1)


def kernel(inputs, embedding, embedding_unnormalized):
    quant, idx3, loss_sum = pl.pallas_call(
        _vq_kernel,
        grid=(B // BB,),
        in_specs=[
            pl.BlockSpec((BB, D, T), lambda g: (g, 0, 0)),
            pl.BlockSpec((K, D), lambda g: (0, 0)),
            pl.BlockSpec((K, D), lambda g: (0, 0)),
        ],
        out_specs=[
            pl.BlockSpec((BB, D, T), lambda g: (g, 0, 0)),
            pl.BlockSpec((BB, 1, T), lambda g: (g, 0, 0)),
            pl.BlockSpec((1, 1), lambda g: (0, 0)),
        ],
        out_shape=[
            jax.ShapeDtypeStruct((B, D, T), jnp.float32),
            jax.ShapeDtypeStruct((B, 1, T), jnp.int32),
            jax.ShapeDtypeStruct((1, 1), jnp.float32),
        ],
        scratch_shapes=[pltpu.VMEM((K, 2 * D), jnp.bfloat16)],
    )(inputs, embedding, embedding_unnormalized)

    loss = loss_sum[0, 0] / jnp.float32(N * D)
    encoding_indices = idx3.reshape(N)
    return (quant, loss, loss, encoding_indices)


# speedup vs baseline: 5.2995x; 1.2164x over previous
import jax
import jax.numpy as jnp
from jax import lax
from jax.experimental import pallas as pl
from jax.experimental.pallas import tpu as pltpu

B, D, T = 32, 64, 576
K = 1024
N = B * T
EPS = 1e-12
BB = 4
W = BB * T


def _vq_kernel(x_ref, emb_ref, embu_ref, quant_ref, idx_ref, loss_ref, cat_ref):
    g = pl.program_id(0)

    @pl.when(g == 0)
    def _init():
        loss_ref[...] = jnp.zeros((1, 1), jnp.float32)
        cat_ref[:, :D] = emb_ref[...].astype(jnp.bfloat16)
        cat_ref[:, D:] = embu_ref[...].astype(jnp.bfloat16)

    x = jnp.concatenate([x_ref[i] for i in range(BB)], axis=1)
    emb = emb_ref[...]

    norm = jnp.sqrt(jnp.sum(x * x, axis=0, keepdims=True))
    xn = x / jnp.maximum(norm, EPS)

    dist = lax.dot_general(
        emb, xn, (((1,), (0,)), ((), ())),
        preferred_element_type=jnp.float32,
    )

    maxval = jnp.max(dist, axis=0, keepdims=True)
    iota_f = lax.broadcasted_iota(jnp.int32, (K, W), 0).astype(jnp.float32)
    idxf = jnp.min(jnp.where(dist >= maxval, iota_f, float(K)), axis=0)

    onehot = (iota_f == idxf[None, :]).astype(jnp.bfloat16)
    combo = lax.dot_general(
        cat_ref[...], onehot, (((0,), (0,)), ((), ())),
        preferred_element_type=jnp.float32,
    )

    idx = idxf.astype(jnp.int32)
    for i in range(BB):
        idx_ref[i, 0] = idx[i * T:(i + 1) * T]
        quant_ref[i] = combo[:D, i * T:(i + 1) * T]
    diff = x - combo[D:]
    loss_ref[...] += jnp.sum(diff * diff).reshape(1, 1)


def kernel(inputs, embedding, embedding_unnormalized):
    quant, idx3, loss_sum = pl.pallas_call(
        _vq_kernel,
        grid=(B // BB,),
        in_specs=[
            pl.BlockSpec((BB, D, T), lambda g: (g, 0, 0)),
            pl.BlockSpec((K, D), lambda g: (0, 0)),
            pl.BlockSpec((K, D), lambda g: (0, 0)),
        ],
        out_specs=[
            pl.BlockSpec((BB, D, T), lambda g: (g, 0, 0)),
            pl.BlockSpec((BB, 1, T), lambda g: (g, 0, 0)),
            pl.BlockSpec((1, 1), lambda g: (0, 0)),
        ],
        out_shape=[
            jax.ShapeDtypeStruct((B, D, T), jnp.float32),
            jax.ShapeDtypeStruct((B, 1, T), jnp.int32),
            jax.ShapeDtypeStruct((1, 1), jnp.float32),
        ],
        scratch_shapes=[pltpu.VMEM((K, 2 * D), jnp.bfloat16)],
    )(inputs, embedding, embedding_unnormalized)

    loss = loss_sum[0, 0] / jnp.float32(N * D)
    encoding_indices = idx3.reshape(N)
    return (quant, loss, loss, encoding_indices)


# speedup vs baseline: 5.4573x; 1.0298x over previous
import jax
import jax.numpy as jnp
from jax import lax
from jax.experimental import pallas as pl
from jax.experimental.pallas import tpu as pltpu

B, D, T = 32, 64, 576
K = 1024
N = B * T
EPS = 1e-12
BB = 8
W = BB * T


def _vq_kernel(x_ref, emb_ref, embu_ref, quant_ref, idx_ref, loss_ref, cat_ref):
    g = pl.program_id(0)

    @pl.when(g == 0)
    def _init():
        loss_ref[...] = jnp.zeros((1, 1), jnp.float32)
        cat_ref[:, :D] = emb_ref[...].astype(jnp.bfloat16)
        cat_ref[:, D:] = embu_ref[...].astype(jnp.bfloat16)

    x = jnp.concatenate([x_ref[i] for i in range(BB)], axis=1)
    emb = emb_ref[...]

    norm = jnp.sqrt(jnp.sum(x * x, axis=0, keepdims=True))
    xn = x / jnp.maximum(norm, EPS)

    dist = lax.dot_general(
        emb, xn, (((1,), (0,)), ((), ())),
        preferred_element_type=jnp.float32,
    )

    maxval = jnp.max(dist, axis=0, keepdims=True)
    iota_f = lax.broadcasted_iota(jnp.int32, (K, W), 0).astype(jnp.float32)
    idxf = jnp.min(jnp.where(dist >= maxval, iota_f, float(K)), axis=0)

    onehot = (iota_f == idxf[None, :]).astype(jnp.bfloat16)
    combo = lax.dot_general(
        cat_ref[...], onehot, (((0,), (0,)), ((), ())),
        preferred_element_type=jnp.float32,
    )

    idx = idxf.astype(jnp.int32)
    for i in range(BB):
        idx_ref[i, 0] = idx[i * T:(i + 1) * T]
        quant_ref[i] = combo[:D, i * T:(i + 1) * T]
    diff = x - combo[D:]
    loss_ref[...] += jnp.sum(diff * diff).reshape(1, 1)


def kernel(inputs, embedding, embedding_unnormalized):
    quant, idx3, loss_sum = pl.pallas_call(
        _vq_kernel,
        grid=(B // BB,),
        in_specs=[
            pl.BlockSpec((BB, D, T), lambda g: (g, 0, 0)),
            pl.BlockSpec((K, D), lambda g: (0, 0)),
            pl.BlockSpec((K, D), lambda g: (0, 0)),
        ],
        out_specs=[
            pl.BlockSpec((BB, D, T), lambda g: (g, 0, 0)),
            pl.BlockSpec((BB, 1, T), lambda g: (g, 0, 0)),
            pl.BlockSpec((1, 1), lambda g: (0, 0)),
        ],
        out_shape=[
            jax.ShapeDtypeStruct((B, D, T), jnp.float32),
            jax.ShapeDtypeStruct((B, 1, T), jnp.int32),
            jax.ShapeDtypeStruct((1, 1), jnp.float32),
        ],
        scratch_shapes=[pltpu.VMEM((K, 2 * D), jnp.bfloat16)],
    )(inputs, embedding, embedding_unnormalized)

    loss = loss_sum[0, 0] / jnp.float32(N * D)
    encoding_indices = idx3.reshape(N)
    return (quant, loss, loss, encoding_indices)
